# Initial kernel scaffold; baseline (speedup 1.0000x reference)
import jax
import jax.numpy as jnp
from jax.experimental import pallas as pl


def kernel(logits, A_hat_indices, A_hat_values):
    raise NotImplementedError("write your pallas kernel here")


# speedup vs baseline: 5.7579x; 5.7579x over previous
import functools

import jax
import jax.numpy as jnp
from jax import lax
from jax.experimental import pallas as pl
from jax.experimental.pallas import tpu as pltpu
from jax.experimental.pallas import tpu_sc as plsc

N = 10000
C = 128
E = 320000
ALPHA = 0.15
NPROP = 5

NC = 2
NS = 16
NW = NC * NS
CHUNK = 128
NCHUNK = 80
EPW = CHUNK * NCHUNK
EPAD = EPW * NW
RPT = 640
RPT_LAST = N - 15 * RPT
LANES = 16


def _spmm_body(logits_hbm, row_hbm, col_hbm, val_hbm, out_hbm,
               col_buf, row_buf, val_buf, rows_buf, agg, sem):
    cid = lax.axis_index("c")
    sid = lax.axis_index("s")
    wid = sid * NC + cid

    pltpu.sync_copy(col_hbm.at[wid], col_buf)
    pltpu.sync_copy(row_hbm.at[wid], row_buf)
    pltpu.sync_copy(val_hbm.at[wid], val_buf)

    def zrow(k, carry):
        for j in range(C // LANES):
            rows_buf[k, pl.ds(j * LANES, LANES)] = jnp.zeros((LANES,), jnp.float32)
        return carry
    lax.fori_loop(0, CHUNK, zrow, 0)
    base_row = sid * RPT

    @pl.when(sid < NS - 1)
    def _zero_main():
        for i in range(RPT // CHUNK):
            pltpu.sync_copy(rows_buf, agg.at[pl.ds(base_row + i * CHUNK, CHUNK)])

    @pl.when(sid == NS - 1)
    def _zero_last():
        for i in range(RPT_LAST // CHUNK):
            pltpu.sync_copy(rows_buf, agg.at[pl.ds(base_row + i * CHUNK, CHUNK)])
        rem = RPT_LAST % CHUNK
        pltpu.sync_copy(rows_buf.at[pl.ds(0, rem)],
                        agg.at[pl.ds(base_row + RPT_LAST - rem, rem)])
    plsc.subcore_barrier()

    def chunk_body(j, carry):
        base = j * CHUNK
        pltpu.async_copy(logits_hbm.at[col_buf.at[j]], rows_buf, sem).wait()

        def scale(k, inner):
            vidx = jnp.full((LANES,), base + k, jnp.int32)
            vs = plsc.load_gather(val_buf, [vidx])
            for jj in range(C // LANES):
                sl = pl.ds(jj * LANES, LANES)
                rows_buf[k, sl] = rows_buf[k, sl] * vs
            return inner
        lax.fori_loop(0, CHUNK, scale, 0)

        pltpu.sync_copy(rows_buf, agg.at[row_buf.at[j]], add=True)
        return carry
    lax.fori_loop(0, NCHUNK, chunk_body, 0)

    plsc.subcore_barrier()

    @pl.when(sid < NS - 1)
    def _out_main():
        sl = pl.ds(base_row, RPT)
        pltpu.sync_copy(agg.at[sl], out_hbm.at[cid, sl])

    @pl.when(sid == NS - 1)
    def _out_last():
        sl = pl.ds(base_row, RPT_LAST)
        pltpu.sync_copy(agg.at[sl], out_hbm.at[cid, sl])


_spmm = functools.partial(
    pl.kernel,
    _spmm_body,
    out_type=jax.ShapeDtypeStruct((NC, N, C), jnp.float32),
    mesh=plsc.VectorSubcoreMesh(core_axis_name="c", subcore_axis_name="s"),
    compiler_params=pltpu.CompilerParams(needs_layout_passes=False),
    scratch_types=[
        pltpu.VMEM((NCHUNK, CHUNK), jnp.int32),
        pltpu.VMEM((NCHUNK, CHUNK), jnp.int32),
        pltpu.VMEM((EPW,), jnp.float32),
        pltpu.VMEM((CHUNK, C), jnp.float32),
        pltpu.VMEM_SHARED((N, C), jnp.float32),
        pltpu.SemaphoreType.DMA,
    ],
)()


def kernel(logits, A_hat_indices, A_hat_values):
    pad = EPAD - E
    pad_idx = (jnp.arange(pad, dtype=jnp.int32) * 37) % N
    row = jnp.concatenate([A_hat_indices[0], pad_idx]).reshape(NW, NCHUNK, CHUNK)
    col = jnp.concatenate([A_hat_indices[1], pad_idx]).reshape(NW, NCHUNK, CHUNK)
    val = jnp.concatenate([A_hat_values, jnp.zeros((pad,), jnp.float32)]
                          ).reshape(NW, EPW)

    x = logits
    for _ in range(NPROP):
        p = _spmm(x, row, col, val)
        x = ALPHA * logits + (1.0 - ALPHA) * (p[0] + p[1])
    return x
